# baseline (speedup 1.0000x reference)
import functools

import jax
import jax.numpy as jnp
from jax.experimental import pallas as pl
from jax.experimental.pallas import tpu as pltpu

_K = 5
_TOPN = _K + 1
_PAD = 8
_NEG = float("-inf")


def _extract_top(cat, n):
    cat_cols = jax.lax.broadcasted_iota(jnp.int32, cat.shape, 1)
    big = jnp.int32(2**31 - 1)
    outs = []
    for _ in range(n):
        v = jnp.max(cat, axis=1, keepdims=True)
        hit = cat == v
        idx = jnp.min(jnp.where(hit, cat_cols, big), axis=1, keepdims=True)
        cat = jnp.where(cat_cols == idx, _NEG, cat)
        outs.append(v)
    return jnp.concatenate(outs, axis=1)


def _body(y_ref, x_ref, out_ref, *, rows, nj, cblk, total_rows):
    i = pl.program_id(0)
    y = y_ref[...]

    def step(j, carry):
        m, s, t, top = carry
        x = x_ref[:, j, :]
        cols = jax.lax.broadcasted_iota(jnp.int32, x.shape, 1) + j * cblk
        ist = cols == y

        t = jnp.maximum(
            t, jnp.max(jnp.where(ist, x, _NEG), axis=1, keepdims=True))

        m_new = jnp.maximum(m, jnp.max(x, axis=1, keepdims=True))
        s = (s * jnp.exp(m - m_new)
             + jnp.sum(jnp.exp(x - m_new), axis=1, keepdims=True))

        xm = jnp.where(ist, _NEG, x)
        cat = jnp.concatenate([top, xm], axis=1)
        newtop = _extract_top(cat, _TOPN)
        top = jnp.concatenate(
            [newtop, jnp.full((rows, _PAD - _TOPN), _NEG, jnp.float32)],
            axis=1)
        return m_new, s, t, top

    init = (jnp.full((rows, 1), _NEG, jnp.float32),
            jnp.zeros((rows, 1), jnp.float32),
            jnp.full((rows, 1), _NEG, jnp.float32),
            jnp.full((rows, _PAD), _NEG, jnp.float32))
    m, s, t, top = jax.lax.fori_loop(0, nj, step, init)

    pp = jnp.exp(t - m) / s
    pn = jnp.exp(top[:, :_TOPN] - m) / s
    loss = jnp.sum((1.0 + pn - pp) ** 2, axis=1, keepdims=True) / _K
    part = (jnp.sum(loss) / total_rows).reshape(1, 1)

    @pl.when(i == 0)
    def _init_out():
        out_ref[...] = jnp.zeros((1, 1), jnp.float32)

    out_ref[...] += part


@functools.partial(jax.jit, static_argnames=("cblk", "rblk"))
def _run(pred, y2, cblk, rblk):
    rows, nclass = pred.shape
    nj = nclass // cblk
    pred3 = pred.reshape(rows, nj, cblk)
    grid = (rows // rblk,)
    body = functools.partial(_body, rows=rblk, nj=nj, cblk=cblk,
                             total_rows=rows)
    out = pl.pallas_call(
        body,
        grid=grid,
        in_specs=[
            pl.BlockSpec((rblk, 1), lambda i: (i, 0)),
            pl.BlockSpec((rblk, nj, cblk), lambda i: (i, 0, 0)),
        ],
        out_specs=pl.BlockSpec((1, 1), lambda i: (0, 0)),
        out_shape=jax.ShapeDtypeStruct((1, 1), jnp.float32),
        compiler_params=pltpu.CompilerParams(
            dimension_semantics=("arbitrary",)),
    )(y2, pred3)
    return out[0, 0]


def kernel(pred, y, epoch=0):
    rows, nclass = pred.shape
    for cand in (2500, 2000, 1250, 1000, 625, 500, 250, 200, 125, 100, 50,
                 25, 20, 10, 8, 5, 4, 2, 1):
        if nclass % cand == 0:
            cblk = cand
            break
    rblk = 8 if rows % 8 == 0 else rows
    y2 = y.reshape(rows, 1).astype(jnp.int32)
    return _run(pred, y2, cblk, rblk)


# speedup vs baseline: 2.6537x; 2.6537x over previous
import functools

import jax
import jax.numpy as jnp
from jax.experimental import pallas as pl
from jax.experimental.pallas import tpu as pltpu

_K = 5
_TOPN = _K + 1
_NEG = float("-inf")


def _extract_top(cat, n):
    cat_cols = jax.lax.broadcasted_iota(jnp.int32, cat.shape, 1)
    big = jnp.int32(2**31 - 1)
    outs = []
    for _ in range(n):
        v = jnp.max(cat, axis=1, keepdims=True)
        hit = cat == v
        idx = jnp.min(jnp.where(hit, cat_cols, big), axis=1, keepdims=True)
        cat = jnp.where(cat_cols == idx, _NEG, cat)
        outs.append(v)
    return jnp.concatenate(outs, axis=1)


def _body(yhi_ref, ylo_ref, x_ref, out_ref, *, rows, nf, w, total_rows):
    i = pl.program_id(0)
    yhi = yhi_ref[...]
    ylo = ylo_ref[...]

    xb = x_ref[...]
    m = jnp.max(jnp.max(xb, axis=2), axis=1).reshape(rows, 1)
    m3 = m.reshape(rows, 1, 1)
    s = jnp.sum(jnp.sum(jnp.exp(xb - m3), axis=2), axis=1).reshape(rows, 1)

    lane = jax.lax.broadcasted_iota(jnp.int32, (rows, w), 1)
    lane_is_t = lane == ylo

    def step(j, carry):
        tacc = carry[0]
        tup = list(carry[1:])
        xj = x_ref[:, j, :]
        ist = lane_is_t & (j == yhi)
        tacc = tacc + jnp.where(ist, xj, 0.0)
        v = jnp.where(ist, _NEG, xj)
        for kk in range(_TOPN):
            hi = jnp.maximum(tup[kk], v)
            v = jnp.minimum(tup[kk], v)
            tup[kk] = hi
        return (tacc, *tup)

    init = (jnp.zeros((rows, w), jnp.float32),) + tuple(
        jnp.full((rows, w), _NEG, jnp.float32) for _ in range(_TOPN))
    res = jax.lax.fori_loop(0, nf, step, init)
    t = jnp.sum(res[0], axis=1, keepdims=True)
    cand = jnp.concatenate(res[1:], axis=1)
    top = _extract_top(cand, _TOPN)

    pp = jnp.exp(t - m) / s
    pn = jnp.exp(top - m) / s
    loss = jnp.sum((1.0 + pn - pp) ** 2, axis=1, keepdims=True) / _K
    part = (jnp.sum(loss) / total_rows).reshape(1, 1)

    @pl.when(i == 0)
    def _init_out():
        out_ref[...] = jnp.zeros((1, 1), jnp.float32)

    out_ref[...] += part


@functools.partial(jax.jit, static_argnames=("w", "rblk"))
def _run(pred, y2, w, rblk):
    rows, nclass = pred.shape
    nf = nclass // w
    pred3 = pred.reshape(rows, nf, w)
    yhi = y2 // w
    ylo = y2 % w
    body = functools.partial(_body, rows=rblk, nf=nf, w=w, total_rows=rows)
    out = pl.pallas_call(
        body,
        grid=(rows // rblk,),
        in_specs=[
            pl.BlockSpec((rblk, 1), lambda i: (i, 0)),
            pl.BlockSpec((rblk, 1), lambda i: (i, 0)),
            pl.BlockSpec((rblk, nf, w), lambda i: (i, 0, 0)),
        ],
        out_specs=pl.BlockSpec((1, 1), lambda i: (0, 0)),
        out_shape=jax.ShapeDtypeStruct((1, 1), jnp.float32),
        compiler_params=pltpu.CompilerParams(
            dimension_semantics=("arbitrary",)),
    )(yhi, ylo, pred3)
    return out[0, 0]


def kernel(pred, y, epoch=0):
    rows, nclass = pred.shape
    for cand in (500, 625, 400, 250, 200, 125, 100, 50, 25, 20, 10, 8, 5,
                 4, 2, 1):
        if nclass % cand == 0:
            w = cand
            break
    rblk = 8 if rows % 8 == 0 else rows
    y2 = y.reshape(rows, 1).astype(jnp.int32)
    return _run(pred, y2, w, rblk)


# speedup vs baseline: 3.4078x; 1.2842x over previous
import functools

import jax
import jax.numpy as jnp
from jax.experimental import pallas as pl
from jax.experimental.pallas import tpu as pltpu

_K = 5
_TOPN = _K + 1
_DEPTH = _K + 2
_NEG = float("-inf")


def _extract_top(cat, n):
    cat_cols = jax.lax.broadcasted_iota(jnp.int32, cat.shape, 1)
    big = jnp.int32(2**31 - 1)
    outs = []
    for _ in range(n):
        v = jnp.max(cat, axis=1, keepdims=True)
        hit = cat == v
        idx = jnp.min(jnp.where(hit, cat_cols, big), axis=1, keepdims=True)
        cat = jnp.where(cat_cols == idx, _NEG, cat)
        outs.append(v)
    return jnp.concatenate(outs, axis=1)


def _insert(tup, v):
    for kk in range(len(tup)):
        hi = jnp.maximum(tup[kk], v)
        v = jnp.minimum(tup[kk], v)
        tup[kk] = hi
    return tup


def _body(yhi_ref, ylo_ref, x_ref, out_ref, *, rows, nf, w, total_rows):
    i = pl.program_id(0)
    ylo = ylo_ref[...]

    def step(j, carry):
        ta = list(carry[:_DEPTH])
        tb = list(carry[_DEPTH:])
        ta = _insert(ta, x_ref[:, 4 * j, :])
        ta = _insert(ta, x_ref[:, 4 * j + 1, :])
        tb = _insert(tb, x_ref[:, 4 * j + 2, :])
        tb = _insert(tb, x_ref[:, 4 * j + 3, :])
        return (*ta, *tb)

    init = tuple(jnp.full((rows, w), _NEG, jnp.float32)
                 for _ in range(2 * _DEPTH))
    res = jax.lax.fori_loop(0, nf // 4, step, init)

    cand = jnp.concatenate(res, axis=1)
    top7 = _extract_top(cand, _DEPTH)

    lane = jax.lax.broadcasted_iota(jnp.int32, (rows, w), 1)
    tmat = jnp.concatenate(
        [x_ref[r, yhi_ref[r, 0], :].reshape(1, w) for r in range(rows)],
        axis=0)
    t = jnp.sum(jnp.where(lane == ylo, tmat, 0.0), axis=1, keepdims=True)

    m = top7[:, :1]
    m3 = m.reshape(rows, 1, 1)
    xb = x_ref[...]
    s = jnp.sum(jnp.sum(jnp.exp(xb - m3), axis=2), axis=1).reshape(rows, 1)

    l6 = top7[:, _TOPN:]
    dropval = jnp.where(t >= l6, t, l6)
    cols7 = jax.lax.broadcasted_iota(jnp.int32, (rows, _DEPTH), 1)
    hit = top7 == dropval
    dropidx = jnp.min(jnp.where(hit, cols7, jnp.int32(2**31 - 1)),
                      axis=1, keepdims=True)
    keep = cols7 != dropidx

    pp = jnp.exp(t - m) / s
    pn = jnp.exp(top7 - m) / s
    terms = (1.0 + pn - pp) ** 2
    loss = jnp.sum(jnp.where(keep, terms, 0.0), axis=1, keepdims=True) / _K
    part = (jnp.sum(loss) / total_rows).reshape(1, 1)

    @pl.when(i == 0)
    def _init_out():
        out_ref[...] = jnp.zeros((1, 1), jnp.float32)

    out_ref[...] += part


@functools.partial(jax.jit, static_argnames=("w", "rblk"))
def _run(pred, y2, w, rblk):
    rows, nclass = pred.shape
    nf = nclass // w
    pred3 = pred.reshape(rows, nf, w)
    yhi = y2 // w
    ylo = y2 % w
    body = functools.partial(_body, rows=rblk, nf=nf, w=w, total_rows=rows)
    out = pl.pallas_call(
        body,
        grid=(rows // rblk,),
        in_specs=[
            pl.BlockSpec((rblk, 1), lambda i: (i, 0),
                         memory_space=pltpu.SMEM),
            pl.BlockSpec((rblk, 1), lambda i: (i, 0)),
            pl.BlockSpec((rblk, nf, w), lambda i: (i, 0, 0)),
        ],
        out_specs=pl.BlockSpec((1, 1), lambda i: (0, 0)),
        out_shape=jax.ShapeDtypeStruct((1, 1), jnp.float32),
        compiler_params=pltpu.CompilerParams(
            dimension_semantics=("arbitrary",)),
    )(yhi, ylo, pred3)
    return out[0, 0]


def kernel(pred, y, epoch=0):
    rows, nclass = pred.shape
    for cand in (250, 500, 125, 200, 100, 50, 25, 20, 10, 8, 5, 4, 2, 1):
        if nclass % (cand * 4) == 0:
            w = cand
            break
    else:
        w = nclass
    rblk = 8 if rows % 8 == 0 else rows
    y2 = y.reshape(rows, 1).astype(jnp.int32)
    return _run(pred, y2, w, rblk)
